# baseline (speedup 1.0000x reference)
import functools

import jax
import jax.numpy as jnp
from jax import lax
from jax.experimental import pallas as pl
from jax.experimental.pallas import tpu as pltpu
from jax.experimental.pallas import tpu_sc as plsc

_NUM_CORES = 2
_NUM_SUBCORES = 16
_NUM_WORKERS = _NUM_CORES * _NUM_SUBCORES


@functools.lru_cache(maxsize=None)
def _build(batch, vocab, dim):
    assert batch % _NUM_WORKERS == 0 and vocab % 4 == 0 and dim == 32
    b_per_w = batch // _NUM_WORKERS

    mesh = plsc.VectorSubcoreMesh(core_axis_name="c", subcore_axis_name="s")

    @functools.partial(
        pl.kernel,
        mesh=mesh,
        out_type=jax.ShapeDtypeStruct((batch, dim), jnp.float32),
        scratch_types=[
            pltpu.VMEM((b_per_w,), jnp.int32),
            pltpu.VMEM((b_per_w,), jnp.int32),
            pltpu.VMEM((b_per_w, 4 * dim), jnp.float32),
            pltpu.VMEM((b_per_w, dim), jnp.float32),
            pltpu.SemaphoreType.DMA,
        ],
        compiler_params=pltpu.CompilerParams(use_tc_tiling_on_sc=False),
    )
    def gather_kernel(t4_hbm, idx_hbm, out_hbm, idx_v, blk_v, rows_v, outb, sem):
        wid = lax.axis_index("s") * _NUM_CORES + lax.axis_index("c")
        base = wid * b_per_w

        pltpu.sync_copy(idx_hbm.at[pl.ds(base, b_per_w)], idx_v)

        def compute_blk(g, _):
            v = idx_v[pl.ds(g * 16, 16)]
            blk_v[pl.ds(g * 16, 16)] = jnp.right_shift(v, 2)
            return 0

        lax.fori_loop(0, b_per_w // 16, compute_blk, 0)

        pltpu.async_copy(t4_hbm.at[blk_v], rows_v, sem).wait()

        def extract(g, _):
            v = idx_v[pl.ds(g * 16, 16)]
            col = (v & 3) * dim
            for t in range(16):
                c = col[t]
                r = g * 16 + t
                outb[r, pl.ds(0, 16)] = rows_v[r, pl.ds(c, 16)]
                outb[r, pl.ds(16, 16)] = rows_v[r, pl.ds(c + 16, 16)]
            return 0

        lax.fori_loop(0, b_per_w // 16, extract, 0)

        pltpu.sync_copy(outb, out_hbm.at[pl.ds(base, b_per_w)])

    return gather_kernel


@jax.jit
def kernel(inputs, table):
    batch = inputs.shape[0]
    vocab, dim = table.shape
    idx = inputs.reshape(batch).astype(jnp.int32)
    t4 = table.reshape(vocab // 4, 4 * dim)
    return _build(batch, vocab, dim)(t4, idx)


# speedup vs baseline: 1.0047x; 1.0047x over previous
import functools

import jax
import jax.numpy as jnp
from jax import lax
from jax.experimental import pallas as pl
from jax.experimental.pallas import tpu as pltpu
from jax.experimental.pallas import tpu_sc as plsc

_NUM_CORES = 2
_NUM_SUBCORES = 16
_NUM_WORKERS = _NUM_CORES * _NUM_SUBCORES


@functools.lru_cache(maxsize=None)
def _build(batch, vocab, dim):
    assert batch % _NUM_WORKERS == 0 and vocab % 4 == 0 and dim == 32
    b_per_w = batch // _NUM_WORKERS

    mesh = plsc.VectorSubcoreMesh(core_axis_name="c", subcore_axis_name="s")

    @functools.partial(
        pl.kernel,
        mesh=mesh,
        out_type=jax.ShapeDtypeStruct((batch, dim), jnp.float32),
        scratch_types=[
            pltpu.VMEM((b_per_w,), jnp.int32),
            pltpu.VMEM((b_per_w,), jnp.int32),
            pltpu.VMEM((128, 4 * dim), jnp.float32),
            pltpu.VMEM((b_per_w, dim), jnp.float32),
            pltpu.SemaphoreType.DMA,
        ],
    )
    def gather_kernel(t4_hbm, idx_hbm, out_hbm, idx_v, blk_v, rows_v, outb, sem):
        wid = lax.axis_index("s") * _NUM_CORES + lax.axis_index("c")
        base = wid * b_per_w

        pltpu.sync_copy(idx_hbm.at[pl.ds(base, b_per_w)], idx_v)

        def compute_blk(g, _):
            v = idx_v[pl.ds(g * 16, 16)]
            blk_v[pl.ds(g * 16, 16)] = jnp.right_shift(v, 2)
            return 0

        lax.fori_loop(0, b_per_w // 16, compute_blk, 0)

        for ch in range(b_per_w // 128):
            pltpu.async_copy(
                t4_hbm.at[blk_v.at[pl.ds(ch * 128, 128)]], rows_v, sem
            ).wait()

            def extract(g, _):
                v = idx_v[pl.ds(ch * 128 + g * 16, 16)]
                col = (v & 3) * dim
                for t in range(16):
                    c = col[t]
                    r = g * 16 + t
                    o = ch * 128 + r
                    outb[o, pl.ds(0, 16)] = rows_v[r, pl.ds(c, 16)]
                    outb[o, pl.ds(16, 16)] = rows_v[r, pl.ds(c + 16, 16)]
                return 0

            lax.fori_loop(0, 8, extract, 0)

        pltpu.sync_copy(outb, out_hbm.at[pl.ds(base, b_per_w)])

    return gather_kernel


@jax.jit
def kernel(inputs, table):
    batch = inputs.shape[0]
    vocab, dim = table.shape
    idx = inputs.reshape(batch).astype(jnp.int32)
    t4 = table.reshape(vocab // 4, 4 * dim)
    return _build(batch, vocab, dim)(t4, idx)
